# baseline (speedup 1.0000x reference)
import functools

import jax
import jax.numpy as jnp
from jax import lax
from jax.experimental import pallas as pl
from jax.experimental.pallas import tpu as pltpu
from jax.experimental.pallas import tpu_sc as plsc

BATCH = 16384
EMBED = 32
PACKED = 128
ROWS_PER = 8
VOCAB = 1000000
IDX_CHUNK = 128

REPACK_BLK = 2048
SPAN = ROWS_PER * REPACK_BLK
N_BLOCKS = -(-VOCAB // SPAN)
VQ_PAD = N_BLOCKS * REPACK_BLK


def _make_gather(num_cores: int, num_workers: int):
    b_per_w = BATCH // num_workers
    n_chunks = b_per_w // IDX_CHUNK
    mesh = plsc.VectorSubcoreMesh(core_axis_name="c", subcore_axis_name="s")

    @functools.partial(
        pl.kernel,
        mesh=mesh,
        compiler_params=pltpu.CompilerParams(use_tc_tiling_on_sc=False),
        out_type=jax.ShapeDtypeStruct((BATCH, 2 * PACKED), jnp.float32),
        scratch_types=[
            pltpu.VMEM((n_chunks, IDX_CHUNK), jnp.int32),
            pltpu.VMEM((n_chunks, IDX_CHUNK), jnp.int32),
            pltpu.VMEM((b_per_w // 2, PACKED), jnp.float32),
            pltpu.VMEM((b_per_w // 2, PACKED), jnp.float32),
            pltpu.SemaphoreType.DMA,
        ],
    )
    def gather_kernel(uid_hbm, mid_hbm, utab_hbm, mtab_hbm, out_hbm,
                      uidx_v, midx_v, urows_v, mrows_v, sem):
        wid = lax.axis_index("s") * num_cores + lax.axis_index("c")
        base = wid * b_per_w
        half = b_per_w // 2
        c_per_p = n_chunks // 2
        pltpu.sync_copy(uid_hbm.at[wid], uidx_v)
        pltpu.sync_copy(mid_hbm.at[wid], midx_v)
        for p in range(2):
            copies = []
            for tab_hbm, idx_v, rows_v in ((utab_hbm, uidx_v, urows_v),
                                           (mtab_hbm, midx_v, mrows_v)):
                for j in range(c_per_p):
                    copies.append(pltpu.async_copy(
                        tab_hbm.at[idx_v.at[p * c_per_p + j]],
                        rows_v.at[pl.ds(j * IDX_CHUNK, IDX_CHUNK)], sem))
            for c in copies:
                c.wait()
            row0 = base + p * half
            pltpu.sync_copy(urows_v,
                            out_hbm.at[pl.ds(row0, half), pl.ds(0, PACKED)])
            pltpu.sync_copy(mrows_v,
                            out_hbm.at[pl.ds(row0, half), pl.ds(PACKED, PACKED)])

    return gather_kernel, n_chunks


def _pack_pair(hi, lo):
    hb = lax.bitcast_convert_type(hi, jnp.uint32)
    lb = lax.bitcast_convert_type(lo, jnp.uint32)
    hb = (hb + jnp.uint32(0x8000)) & jnp.uint32(0xFFFF0000)
    lb = (lb + jnp.uint32(0x8000)) >> jnp.uint32(16)
    return lax.bitcast_convert_type(hb | lb, jnp.float32)


def _repack_body(t_ref, eye_ref, o_ref):
    x = t_ref[...]
    n = o_ref.shape[0]
    for half in range(2):
        x4 = jnp.concatenate(
            [x[:, n * (4 * half + a):n * (4 * half + a + 1)] for a in range(4)],
            axis=0)
        y = jax.lax.dot_general(
            x4, eye_ref[...], (((0,), (0,)), ((), ())),
            preferred_element_type=jnp.float32)
        for a in range(4):
            w = 4 * half + a
            o_ref[:, 16 * w:16 * (w + 1)] = _pack_pair(
                y[:, 32 * a + 16:32 * a + 32], y[:, 32 * a:32 * a + 16])


def _repack(table_t, eye):
    return pl.pallas_call(
        _repack_body,
        grid=(N_BLOCKS,),
        in_specs=[
            pl.BlockSpec((EMBED, SPAN), lambda i: (0, i)),
            pl.BlockSpec((128, 128), lambda i: (0, 0)),
        ],
        out_specs=pl.BlockSpec((REPACK_BLK, PACKED), lambda i: (i, 0)),
        out_shape=jax.ShapeDtypeStruct((VQ_PAD, PACKED), jnp.float32),
    )(table_t, eye)


def _select16(rows, rem, off):
    w = [rows[:, off + 16 * a:off + 16 * (a + 1)] for a in range(8)]
    b0 = (rem & 1) == 0
    b1 = (rem & 2) == 0
    l1 = [jnp.where(b0, w[2 * a], w[2 * a + 1]) for a in range(4)]
    l2 = [jnp.where(b1, l1[2 * a], l1[2 * a + 1]) for a in range(2)]
    return jnp.where(rem < 4, l2[0], l2[1])


def _unpack(sel):
    w = lax.bitcast_convert_type(sel, jnp.uint32)
    lo = lax.bitcast_convert_type(w << jnp.uint32(16), jnp.float32)
    hi = lax.bitcast_convert_type(w & jnp.uint32(0xFFFF0000), jnp.float32)
    return lo, hi


def _mlp_body(g_ref, ur_ref, mr_ref, w1a_ref, w1b_ref, w1c_ref, w1d_ref,
              b1_ref, w2_ref, b2_ref, w3_ref, b3_ref, o_ref):
    g = g_ref[...]
    ulo, uhi = _unpack(_select16(g, ur_ref[...], 0))
    mlo, mhi = _unpack(_select16(g, mr_ref[...], PACKED))
    f32 = jnp.float32
    h1 = jnp.dot(ulo, w1a_ref[...], preferred_element_type=f32)
    h1 += jnp.dot(uhi, w1b_ref[...], preferred_element_type=f32)
    h1 += jnp.dot(mlo, w1c_ref[...], preferred_element_type=f32)
    h1 += jnp.dot(mhi, w1d_ref[...], preferred_element_type=f32)
    h1 = jnp.maximum(h1 + b1_ref[...], 0.0)
    h2 = jnp.dot(h1, w2_ref[...], preferred_element_type=f32)
    h2 = jnp.maximum(h2 + b2_ref[...], 0.0)
    o_ref[...] = jnp.sum(h2 * w3_ref[...], axis=1, keepdims=True) + b3_ref[...]


def _mlp_call(g, ur, mr, W1s, b1, W2, b2, W3r, b3, blk: int):
    grid = (BATCH // blk,)
    return pl.pallas_call(
        _mlp_body,
        grid=grid,
        in_specs=[
            pl.BlockSpec((blk, 2 * PACKED), lambda i: (i, 0)),
            pl.BlockSpec((blk, 1), lambda i: (i, 0)),
            pl.BlockSpec((blk, 1), lambda i: (i, 0)),
            pl.BlockSpec((16, 256), lambda i: (0, 0)),
            pl.BlockSpec((16, 256), lambda i: (0, 0)),
            pl.BlockSpec((16, 256), lambda i: (0, 0)),
            pl.BlockSpec((16, 256), lambda i: (0, 0)),
            pl.BlockSpec((1, 256), lambda i: (0, 0)),
            pl.BlockSpec((256, 64), lambda i: (0, 0)),
            pl.BlockSpec((1, 64), lambda i: (0, 0)),
            pl.BlockSpec((1, 64), lambda i: (0, 0)),
            pl.BlockSpec((1, 1), lambda i: (0, 0)),
        ],
        out_specs=pl.BlockSpec((blk, 1), lambda i: (i, 0)),
        out_shape=jax.ShapeDtypeStruct((BATCH, 1), jnp.float32),
    )(g, ur, mr, *W1s, b1, W2, b2, W3r, b3)


def kernel(userId, movieId, user_table, movie_table, W1, b1, W2, b2, W3, b3):
    info = plsc.get_sparse_core_info()
    num_workers = info.num_cores * info.num_subcores
    gather_kernel, n_chunks = _make_gather(info.num_cores, num_workers)

    uid = userId.astype(jnp.int32)
    mid = movieId.astype(jnp.int32)
    uq = (REPACK_BLK * (uid // SPAN) + (uid % SPAN) % REPACK_BLK)
    mq = (REPACK_BLK * (mid // SPAN) + (mid % SPAN) % REPACK_BLK)
    uq = uq.reshape(num_workers, n_chunks, IDX_CHUNK)
    mq = mq.reshape(num_workers, n_chunks, IDX_CHUNK)
    eye = jnp.eye(128, dtype=jnp.float32)
    tab_u = _repack(user_table.T, eye)
    tab_m = _repack(movie_table.T, eye)
    g = gather_kernel(uq, mq, tab_u, tab_m)

    ur = ((uid % SPAN) // REPACK_BLK).reshape(BATCH, 1)
    mr = ((mid % SPAN) // REPACK_BLK).reshape(BATCH, 1)
    W1s = (W1[0:16], W1[16:32], W1[32:48], W1[48:64])
    return _mlp_call(g, ur, mr, W1s, b1.reshape(1, 256), W2,
                     b2.reshape(1, 64), W3.reshape(1, 64), b3.reshape(1, 1),
                     blk=1024)


# speedup vs baseline: 1.6125x; 1.6125x over previous
import functools

import jax
import jax.numpy as jnp
from jax import lax
from jax.experimental import pallas as pl
from jax.experimental.pallas import tpu as pltpu
from jax.experimental.pallas import tpu_sc as plsc

BATCH = 16384
EMBED = 32
PACKED = 128
ROWS_PER = 8
VOCAB = 1000000
IDX_CHUNK = 128

REPACK_BLK = 2048
SPAN = ROWS_PER * REPACK_BLK
N_BLOCKS = -(-VOCAB // SPAN)
VQ_PAD = N_BLOCKS * REPACK_BLK


def _make_gather(num_cores: int, num_workers: int):
    b_per_w = BATCH // num_workers
    n_chunks = b_per_w // IDX_CHUNK
    mesh = plsc.VectorSubcoreMesh(core_axis_name="c", subcore_axis_name="s")

    @functools.partial(
        pl.kernel,
        mesh=mesh,
        compiler_params=pltpu.CompilerParams(use_tc_tiling_on_sc=False),
        out_type=jax.ShapeDtypeStruct((BATCH, 2 * PACKED), jnp.float32),
        scratch_types=[
            pltpu.VMEM((n_chunks, IDX_CHUNK), jnp.int32),
            pltpu.VMEM((n_chunks, IDX_CHUNK), jnp.int32),
            pltpu.VMEM((b_per_w // 2, PACKED), jnp.float32),
            pltpu.VMEM((b_per_w // 2, PACKED), jnp.float32),
            pltpu.SemaphoreType.DMA,
        ],
    )
    def gather_kernel(uid_hbm, mid_hbm, utab_hbm, mtab_hbm, out_hbm,
                      uidx_v, midx_v, urows_v, mrows_v, sem):
        wid = lax.axis_index("s") * num_cores + lax.axis_index("c")
        base = wid * b_per_w
        half = b_per_w // 2
        c_per_p = n_chunks // 2
        pltpu.sync_copy(uid_hbm.at[wid], uidx_v)
        pltpu.sync_copy(mid_hbm.at[wid], midx_v)
        for p in range(2):
            copies = []
            for tab_hbm, idx_v, rows_v in ((utab_hbm, uidx_v, urows_v),
                                           (mtab_hbm, midx_v, mrows_v)):
                for j in range(c_per_p):
                    copies.append(pltpu.async_copy(
                        tab_hbm.at[idx_v.at[p * c_per_p + j]],
                        rows_v.at[pl.ds(j * IDX_CHUNK, IDX_CHUNK)], sem))
            for c in copies:
                c.wait()
            row0 = base + p * half
            pltpu.sync_copy(urows_v,
                            out_hbm.at[pl.ds(row0, half), pl.ds(0, PACKED)])
            pltpu.sync_copy(mrows_v,
                            out_hbm.at[pl.ds(row0, half), pl.ds(PACKED, PACKED)])

    return gather_kernel, n_chunks


def _pack_pair(hi, lo):
    hb = lax.bitcast_convert_type(hi, jnp.uint32)
    lb = lax.bitcast_convert_type(lo, jnp.uint32)
    hb = (hb + jnp.uint32(0x8000)) & jnp.uint32(0xFFFF0000)
    lb = (lb + jnp.uint32(0x8000)) >> jnp.uint32(16)
    return lax.bitcast_convert_type(hb | lb, jnp.float32)


def _repack_body(t_ref, phi_ref, plo_ref, o_ref):
    x = t_ref[...]
    n = o_ref.shape[0]
    x8 = jnp.concatenate([x[:, n * a:n * (a + 1)] for a in range(8)],
                         axis=0)
    y_hi = jax.lax.dot_general(
        x8, phi_ref[...], (((0,), (0,)), ((), ())),
        preferred_element_type=jnp.float32)
    y_lo = jax.lax.dot_general(
        x8, plo_ref[...], (((0,), (0,)), ((), ())),
        preferred_element_type=jnp.float32)
    o_ref[...] = _pack_pair(y_hi, y_lo)


def _repack(table_t, phi, plo):
    return pl.pallas_call(
        _repack_body,
        grid=(N_BLOCKS,),
        in_specs=[
            pl.BlockSpec((EMBED, SPAN), lambda i: (0, i)),
            pl.BlockSpec((256, 128), lambda i: (0, 0)),
            pl.BlockSpec((256, 128), lambda i: (0, 0)),
        ],
        out_specs=pl.BlockSpec((REPACK_BLK, PACKED), lambda i: (i, 0)),
        out_shape=jax.ShapeDtypeStruct((VQ_PAD, PACKED), jnp.float32),
    )(table_t, phi, plo)


def _select16(rows, rem, off):
    w = [rows[:, off + 16 * a:off + 16 * (a + 1)] for a in range(8)]
    b0 = (rem & 1) == 0
    b1 = (rem & 2) == 0
    l1 = [jnp.where(b0, w[2 * a], w[2 * a + 1]) for a in range(4)]
    l2 = [jnp.where(b1, l1[2 * a], l1[2 * a + 1]) for a in range(2)]
    return jnp.where(rem < 4, l2[0], l2[1])


def _unpack(sel):
    w = lax.bitcast_convert_type(sel, jnp.uint32)
    lo = lax.bitcast_convert_type(w << jnp.uint32(16), jnp.float32)
    hi = lax.bitcast_convert_type(w & jnp.uint32(0xFFFF0000), jnp.float32)
    return lo, hi


def _mlp_body(g_ref, ur_ref, mr_ref, w1a_ref, w1b_ref, w1c_ref, w1d_ref,
              b1_ref, w2_ref, b2_ref, w3_ref, b3_ref, o_ref):
    g = g_ref[...]
    ulo, uhi = _unpack(_select16(g, ur_ref[...], 0))
    mlo, mhi = _unpack(_select16(g, mr_ref[...], PACKED))
    f32 = jnp.float32
    h1 = jnp.dot(ulo, w1a_ref[...], preferred_element_type=f32)
    h1 += jnp.dot(uhi, w1b_ref[...], preferred_element_type=f32)
    h1 += jnp.dot(mlo, w1c_ref[...], preferred_element_type=f32)
    h1 += jnp.dot(mhi, w1d_ref[...], preferred_element_type=f32)
    h1 = jnp.maximum(h1 + b1_ref[...], 0.0)
    h2 = jnp.dot(h1, w2_ref[...], preferred_element_type=f32)
    h2 = jnp.maximum(h2 + b2_ref[...], 0.0)
    o_ref[...] = jnp.sum(h2 * w3_ref[...], axis=1, keepdims=True) + b3_ref[...]


def _mlp_call(g, ur, mr, W1s, b1, W2, b2, W3r, b3, blk: int):
    grid = (BATCH // blk,)
    return pl.pallas_call(
        _mlp_body,
        grid=grid,
        in_specs=[
            pl.BlockSpec((blk, 2 * PACKED), lambda i: (i, 0)),
            pl.BlockSpec((blk, 1), lambda i: (i, 0)),
            pl.BlockSpec((blk, 1), lambda i: (i, 0)),
            pl.BlockSpec((16, 256), lambda i: (0, 0)),
            pl.BlockSpec((16, 256), lambda i: (0, 0)),
            pl.BlockSpec((16, 256), lambda i: (0, 0)),
            pl.BlockSpec((16, 256), lambda i: (0, 0)),
            pl.BlockSpec((1, 256), lambda i: (0, 0)),
            pl.BlockSpec((256, 64), lambda i: (0, 0)),
            pl.BlockSpec((1, 64), lambda i: (0, 0)),
            pl.BlockSpec((1, 64), lambda i: (0, 0)),
            pl.BlockSpec((1, 1), lambda i: (0, 0)),
        ],
        out_specs=pl.BlockSpec((blk, 1), lambda i: (i, 0)),
        out_shape=jax.ShapeDtypeStruct((BATCH, 1), jnp.float32),
    )(g, ur, mr, *W1s, b1, W2, b2, W3r, b3)


def kernel(userId, movieId, user_table, movie_table, W1, b1, W2, b2, W3, b3):
    info = plsc.get_sparse_core_info()
    num_workers = info.num_cores * info.num_subcores
    gather_kernel, n_chunks = _make_gather(info.num_cores, num_workers)

    uid = userId.astype(jnp.int32)
    mid = movieId.astype(jnp.int32)
    uq = (REPACK_BLK * (uid // SPAN) + (uid % SPAN) % REPACK_BLK)
    mq = (REPACK_BLK * (mid // SPAN) + (mid % SPAN) % REPACK_BLK)
    uq = uq.reshape(num_workers, n_chunks, IDX_CHUNK)
    mq = mq.reshape(num_workers, n_chunks, IDX_CHUNK)
    k_idx = lax.broadcasted_iota(jnp.int32, (256, 128), 0)
    c_idx = lax.broadcasted_iota(jnp.int32, (256, 128), 1)
    src = 32 * (c_idx // 16) + (c_idx % 16)
    phi = (k_idx == src + 16).astype(jnp.float32)
    plo = (k_idx == src).astype(jnp.float32)
    tab_u = _repack(user_table.T, phi, plo)
    tab_m = _repack(movie_table.T, phi, plo)
    g = gather_kernel(uq, mq, tab_u, tab_m)

    ur = ((uid % SPAN) // REPACK_BLK).reshape(BATCH, 1)
    mr = ((mid % SPAN) // REPACK_BLK).reshape(BATCH, 1)
    W1s = (W1[0:16], W1[16:32], W1[32:48], W1[48:64])
    return _mlp_call(g, ur, mr, W1s, b1.reshape(1, 256), W2,
                     b2.reshape(1, 64), W3.reshape(1, 64), b3.reshape(1, 1),
                     blk=1024)


# speedup vs baseline: 1.8789x; 1.1652x over previous
import functools

import jax
import jax.numpy as jnp
from jax import lax
from jax.experimental import pallas as pl
from jax.experimental.pallas import tpu as pltpu
from jax.experimental.pallas import tpu_sc as plsc

BATCH = 16384
EMBED = 32
PACKED = 128
ROWS_PER = 8
VOCAB = 1000000
IDX_CHUNK = 128

REPACK_BLK = 4096
SPAN = ROWS_PER * REPACK_BLK
N_BLOCKS = -(-VOCAB // SPAN)
VQ_PAD = N_BLOCKS * REPACK_BLK


def _make_gather(num_cores: int, num_workers: int):
    b_per_w = BATCH // num_workers
    n_chunks = b_per_w // IDX_CHUNK
    mesh = plsc.VectorSubcoreMesh(core_axis_name="c", subcore_axis_name="s")

    @functools.partial(
        pl.kernel,
        mesh=mesh,
        compiler_params=pltpu.CompilerParams(use_tc_tiling_on_sc=False),
        out_type=jax.ShapeDtypeStruct((BATCH, 2 * PACKED), jnp.float32),
        scratch_types=[
            pltpu.VMEM((n_chunks, IDX_CHUNK), jnp.int32),
            pltpu.VMEM((n_chunks, IDX_CHUNK), jnp.int32),
            pltpu.VMEM((b_per_w // 2, PACKED), jnp.float32),
            pltpu.VMEM((b_per_w // 2, PACKED), jnp.float32),
            pltpu.SemaphoreType.DMA,
        ],
    )
    def gather_kernel(uid_hbm, mid_hbm, utab_hbm, mtab_hbm, out_hbm,
                      uidx_v, midx_v, urows_v, mrows_v, sem):
        wid = lax.axis_index("s") * num_cores + lax.axis_index("c")
        base = wid * b_per_w
        half = b_per_w // 2
        c_per_p = n_chunks // 2
        pltpu.sync_copy(uid_hbm.at[wid], uidx_v)
        pltpu.sync_copy(mid_hbm.at[wid], midx_v)
        for p in range(2):
            copies = []
            for tab_hbm, idx_v, rows_v in ((utab_hbm, uidx_v, urows_v),
                                           (mtab_hbm, midx_v, mrows_v)):
                for j in range(c_per_p):
                    copies.append(pltpu.async_copy(
                        tab_hbm.at[idx_v.at[p * c_per_p + j]],
                        rows_v.at[pl.ds(j * IDX_CHUNK, IDX_CHUNK)], sem))
            for c in copies:
                c.wait()
            row0 = base + p * half
            pltpu.sync_copy(urows_v,
                            out_hbm.at[pl.ds(row0, half), pl.ds(0, PACKED)])
            pltpu.sync_copy(mrows_v,
                            out_hbm.at[pl.ds(row0, half), pl.ds(PACKED, PACKED)])

    return gather_kernel, n_chunks


def _pack_pair(hi, lo):
    hb = lax.bitcast_convert_type(hi, jnp.uint32)
    lb = lax.bitcast_convert_type(lo, jnp.uint32)
    hb = (hb + jnp.uint32(0x8000)) & jnp.uint32(0xFFFF0000)
    lb = (lb + jnp.uint32(0x8000)) >> jnp.uint32(16)
    return lax.bitcast_convert_type(hb | lb, jnp.float32)


def _repack_body(t_ref, phi_ref, plo_ref, o_ref):
    x = t_ref[...]
    n = o_ref.shape[0]
    x8 = jnp.concatenate([x[:, n * a:n * (a + 1)] for a in range(8)],
                         axis=0)
    y_hi = jax.lax.dot_general(
        x8, phi_ref[...], (((0,), (0,)), ((), ())),
        preferred_element_type=jnp.float32)
    y_lo = jax.lax.dot_general(
        x8, plo_ref[...], (((0,), (0,)), ((), ())),
        preferred_element_type=jnp.float32)
    o_ref[...] = _pack_pair(y_hi, y_lo)


def _repack(table_t, phi, plo):
    return pl.pallas_call(
        _repack_body,
        grid=(N_BLOCKS,),
        in_specs=[
            pl.BlockSpec((EMBED, SPAN), lambda i: (0, i)),
            pl.BlockSpec((256, 128), lambda i: (0, 0)),
            pl.BlockSpec((256, 128), lambda i: (0, 0)),
        ],
        out_specs=pl.BlockSpec((REPACK_BLK, PACKED), lambda i: (i, 0)),
        out_shape=jax.ShapeDtypeStruct((VQ_PAD, PACKED), jnp.float32),
    )(table_t, phi, plo)


def _select16(rows, rem, off):
    w = [rows[:, off + 16 * a:off + 16 * (a + 1)] for a in range(8)]
    b0 = (rem & 1) == 0
    b1 = (rem & 2) == 0
    l1 = [jnp.where(b0, w[2 * a], w[2 * a + 1]) for a in range(4)]
    l2 = [jnp.where(b1, l1[2 * a], l1[2 * a + 1]) for a in range(2)]
    return jnp.where(rem < 4, l2[0], l2[1])


def _unpack(sel):
    w = lax.bitcast_convert_type(sel, jnp.uint32)
    lo = lax.bitcast_convert_type(w << jnp.uint32(16), jnp.float32)
    hi = lax.bitcast_convert_type(w & jnp.uint32(0xFFFF0000), jnp.float32)
    return lo, hi


def _mlp_body(g_ref, ur_ref, mr_ref, w1a_ref, w1b_ref, w1c_ref, w1d_ref,
              b1_ref, w2_ref, b2_ref, w3_ref, b3_ref, o_ref):
    g = g_ref[...]
    ulo, uhi = _unpack(_select16(g, ur_ref[...], 0))
    mlo, mhi = _unpack(_select16(g, mr_ref[...], PACKED))
    f32 = jnp.float32
    h1 = jnp.dot(ulo, w1a_ref[...], preferred_element_type=f32)
    h1 += jnp.dot(uhi, w1b_ref[...], preferred_element_type=f32)
    h1 += jnp.dot(mlo, w1c_ref[...], preferred_element_type=f32)
    h1 += jnp.dot(mhi, w1d_ref[...], preferred_element_type=f32)
    h1 = jnp.maximum(h1 + b1_ref[...], 0.0)
    h2 = jnp.dot(h1, w2_ref[...], preferred_element_type=f32)
    h2 = jnp.maximum(h2 + b2_ref[...], 0.0)
    o_ref[...] = jnp.sum(h2 * w3_ref[...], axis=1, keepdims=True) + b3_ref[...]


def _mlp_call(g, ur, mr, W1s, b1, W2, b2, W3r, b3, blk: int):
    grid = (BATCH // blk,)
    return pl.pallas_call(
        _mlp_body,
        grid=grid,
        in_specs=[
            pl.BlockSpec((blk, 2 * PACKED), lambda i: (i, 0)),
            pl.BlockSpec((blk, 1), lambda i: (i, 0)),
            pl.BlockSpec((blk, 1), lambda i: (i, 0)),
            pl.BlockSpec((16, 256), lambda i: (0, 0)),
            pl.BlockSpec((16, 256), lambda i: (0, 0)),
            pl.BlockSpec((16, 256), lambda i: (0, 0)),
            pl.BlockSpec((16, 256), lambda i: (0, 0)),
            pl.BlockSpec((1, 256), lambda i: (0, 0)),
            pl.BlockSpec((256, 64), lambda i: (0, 0)),
            pl.BlockSpec((1, 64), lambda i: (0, 0)),
            pl.BlockSpec((1, 64), lambda i: (0, 0)),
            pl.BlockSpec((1, 1), lambda i: (0, 0)),
        ],
        out_specs=pl.BlockSpec((blk, 1), lambda i: (i, 0)),
        out_shape=jax.ShapeDtypeStruct((BATCH, 1), jnp.float32),
    )(g, ur, mr, *W1s, b1, W2, b2, W3r, b3)


def kernel(userId, movieId, user_table, movie_table, W1, b1, W2, b2, W3, b3):
    info = plsc.get_sparse_core_info()
    num_workers = info.num_cores * info.num_subcores
    gather_kernel, n_chunks = _make_gather(info.num_cores, num_workers)

    uid = userId.astype(jnp.int32)
    mid = movieId.astype(jnp.int32)
    uq = (REPACK_BLK * (uid // SPAN) + (uid % SPAN) % REPACK_BLK)
    mq = (REPACK_BLK * (mid // SPAN) + (mid % SPAN) % REPACK_BLK)
    uq = uq.reshape(num_workers, n_chunks, IDX_CHUNK)
    mq = mq.reshape(num_workers, n_chunks, IDX_CHUNK)
    k_idx = lax.broadcasted_iota(jnp.int32, (256, 128), 0)
    c_idx = lax.broadcasted_iota(jnp.int32, (256, 128), 1)
    src = 32 * (c_idx // 16) + (c_idx % 16)
    phi = (k_idx == src + 16).astype(jnp.float32)
    plo = (k_idx == src).astype(jnp.float32)
    tab_u = _repack(user_table.T, phi, plo)
    tab_m = _repack(movie_table.T, phi, plo)
    g = gather_kernel(uq, mq, tab_u, tab_m)

    ur = ((uid % SPAN) // REPACK_BLK).reshape(BATCH, 1)
    mr = ((mid % SPAN) // REPACK_BLK).reshape(BATCH, 1)
    W1s = (W1[0:16], W1[16:32], W1[32:48], W1[48:64])
    return _mlp_call(g, ur, mr, W1s, b1.reshape(1, 256), W2,
                     b2.reshape(1, 64), W3.reshape(1, 64), b3.reshape(1, 1),
                     blk=1024)


# speedup vs baseline: 1.9238x; 1.0239x over previous
import functools

import jax
import jax.numpy as jnp
from jax import lax
from jax.experimental import pallas as pl
from jax.experimental.pallas import tpu as pltpu
from jax.experimental.pallas import tpu_sc as plsc

BATCH = 16384
EMBED = 32
PACKED = 128
ROWS_PER = 8
VOCAB = 1000000
IDX_CHUNK = 128

REPACK_BLK = 8192
SPAN = ROWS_PER * REPACK_BLK
N_BLOCKS = -(-VOCAB // SPAN)
VQ_PAD = N_BLOCKS * REPACK_BLK


def _make_gather(num_cores: int, num_workers: int):
    b_per_w = BATCH // num_workers
    n_chunks = b_per_w // IDX_CHUNK
    mesh = plsc.VectorSubcoreMesh(core_axis_name="c", subcore_axis_name="s")

    @functools.partial(
        pl.kernel,
        mesh=mesh,
        compiler_params=pltpu.CompilerParams(use_tc_tiling_on_sc=False),
        out_type=jax.ShapeDtypeStruct((BATCH, 2 * PACKED), jnp.float32),
        scratch_types=[
            pltpu.VMEM((n_chunks, IDX_CHUNK), jnp.int32),
            pltpu.VMEM((n_chunks, IDX_CHUNK), jnp.int32),
            pltpu.VMEM((b_per_w // 2, PACKED), jnp.float32),
            pltpu.VMEM((b_per_w // 2, PACKED), jnp.float32),
            pltpu.SemaphoreType.DMA,
        ],
    )
    def gather_kernel(uid_hbm, mid_hbm, utab_hbm, mtab_hbm, out_hbm,
                      uidx_v, midx_v, urows_v, mrows_v, sem):
        wid = lax.axis_index("s") * num_cores + lax.axis_index("c")
        base = wid * b_per_w
        half = b_per_w // 2
        c_per_p = n_chunks // 2
        pltpu.sync_copy(uid_hbm.at[wid], uidx_v)
        pltpu.sync_copy(mid_hbm.at[wid], midx_v)
        for p in range(2):
            copies = []
            for tab_hbm, idx_v, rows_v in ((utab_hbm, uidx_v, urows_v),
                                           (mtab_hbm, midx_v, mrows_v)):
                for j in range(c_per_p):
                    copies.append(pltpu.async_copy(
                        tab_hbm.at[idx_v.at[p * c_per_p + j]],
                        rows_v.at[pl.ds(j * IDX_CHUNK, IDX_CHUNK)], sem))
            for c in copies:
                c.wait()
            row0 = base + p * half
            pltpu.sync_copy(urows_v,
                            out_hbm.at[pl.ds(row0, half), pl.ds(0, PACKED)])
            pltpu.sync_copy(mrows_v,
                            out_hbm.at[pl.ds(row0, half), pl.ds(PACKED, PACKED)])

    return gather_kernel, n_chunks


def _pack_pair(hi, lo):
    hb = lax.bitcast_convert_type(hi, jnp.uint32)
    lb = lax.bitcast_convert_type(lo, jnp.uint32)
    hb = (hb + jnp.uint32(0x8000)) & jnp.uint32(0xFFFF0000)
    lb = (lb + jnp.uint32(0x8000)) >> jnp.uint32(16)
    return lax.bitcast_convert_type(hb | lb, jnp.float32)


def _repack_body(t_ref, phi_ref, plo_ref, o_ref):
    x = t_ref[...]
    n = o_ref.shape[0]
    x8 = jnp.concatenate([x[:, n * a:n * (a + 1)] for a in range(8)],
                         axis=0)
    y_hi = jax.lax.dot_general(
        x8, phi_ref[...], (((0,), (0,)), ((), ())),
        preferred_element_type=jnp.float32)
    y_lo = jax.lax.dot_general(
        x8, plo_ref[...], (((0,), (0,)), ((), ())),
        preferred_element_type=jnp.float32)
    o_ref[...] = _pack_pair(y_hi, y_lo)


def _repack(table_t, phi, plo):
    return pl.pallas_call(
        _repack_body,
        grid=(N_BLOCKS,),
        in_specs=[
            pl.BlockSpec((EMBED, SPAN), lambda i: (0, i)),
            pl.BlockSpec((256, 128), lambda i: (0, 0)),
            pl.BlockSpec((256, 128), lambda i: (0, 0)),
        ],
        out_specs=pl.BlockSpec((REPACK_BLK, PACKED), lambda i: (i, 0)),
        out_shape=jax.ShapeDtypeStruct((VQ_PAD, PACKED), jnp.float32),
    )(table_t, phi, plo)


def _select16(rows, rem, off):
    w = [rows[:, off + 16 * a:off + 16 * (a + 1)] for a in range(8)]
    b0 = (rem & 1) == 0
    b1 = (rem & 2) == 0
    l1 = [jnp.where(b0, w[2 * a], w[2 * a + 1]) for a in range(4)]
    l2 = [jnp.where(b1, l1[2 * a], l1[2 * a + 1]) for a in range(2)]
    return jnp.where(rem < 4, l2[0], l2[1])


def _unpack(sel):
    w = lax.bitcast_convert_type(sel, jnp.uint32)
    lo = lax.bitcast_convert_type(w << jnp.uint32(16), jnp.float32)
    hi = lax.bitcast_convert_type(w & jnp.uint32(0xFFFF0000), jnp.float32)
    return lo, hi


def _mlp_body(g_ref, ur_ref, mr_ref, w1a_ref, w1b_ref, w1c_ref, w1d_ref,
              b1_ref, w2_ref, b2_ref, w3_ref, b3_ref, o_ref):
    g = g_ref[...]
    ulo, uhi = _unpack(_select16(g, ur_ref[...], 0))
    mlo, mhi = _unpack(_select16(g, mr_ref[...], PACKED))
    f32 = jnp.float32
    h1 = jnp.dot(ulo, w1a_ref[...], preferred_element_type=f32)
    h1 += jnp.dot(uhi, w1b_ref[...], preferred_element_type=f32)
    h1 += jnp.dot(mlo, w1c_ref[...], preferred_element_type=f32)
    h1 += jnp.dot(mhi, w1d_ref[...], preferred_element_type=f32)
    h1 = jnp.maximum(h1 + b1_ref[...], 0.0)
    h2 = jnp.dot(h1, w2_ref[...], preferred_element_type=f32)
    h2 = jnp.maximum(h2 + b2_ref[...], 0.0)
    o_ref[...] = jnp.sum(h2 * w3_ref[...], axis=1, keepdims=True) + b3_ref[...]


def _mlp_call(g, ur, mr, W1s, b1, W2, b2, W3r, b3, blk: int):
    grid = (BATCH // blk,)
    return pl.pallas_call(
        _mlp_body,
        grid=grid,
        in_specs=[
            pl.BlockSpec((blk, 2 * PACKED), lambda i: (i, 0)),
            pl.BlockSpec((blk, 1), lambda i: (i, 0)),
            pl.BlockSpec((blk, 1), lambda i: (i, 0)),
            pl.BlockSpec((16, 256), lambda i: (0, 0)),
            pl.BlockSpec((16, 256), lambda i: (0, 0)),
            pl.BlockSpec((16, 256), lambda i: (0, 0)),
            pl.BlockSpec((16, 256), lambda i: (0, 0)),
            pl.BlockSpec((1, 256), lambda i: (0, 0)),
            pl.BlockSpec((256, 64), lambda i: (0, 0)),
            pl.BlockSpec((1, 64), lambda i: (0, 0)),
            pl.BlockSpec((1, 64), lambda i: (0, 0)),
            pl.BlockSpec((1, 1), lambda i: (0, 0)),
        ],
        out_specs=pl.BlockSpec((blk, 1), lambda i: (i, 0)),
        out_shape=jax.ShapeDtypeStruct((BATCH, 1), jnp.float32),
    )(g, ur, mr, *W1s, b1, W2, b2, W3r, b3)


def kernel(userId, movieId, user_table, movie_table, W1, b1, W2, b2, W3, b3):
    info = plsc.get_sparse_core_info()
    num_workers = info.num_cores * info.num_subcores
    gather_kernel, n_chunks = _make_gather(info.num_cores, num_workers)

    uid = userId.astype(jnp.int32)
    mid = movieId.astype(jnp.int32)
    uq = (REPACK_BLK * (uid // SPAN) + (uid % SPAN) % REPACK_BLK)
    mq = (REPACK_BLK * (mid // SPAN) + (mid % SPAN) % REPACK_BLK)
    uq = uq.reshape(num_workers, n_chunks, IDX_CHUNK)
    mq = mq.reshape(num_workers, n_chunks, IDX_CHUNK)
    k_idx = lax.broadcasted_iota(jnp.int32, (256, 128), 0)
    c_idx = lax.broadcasted_iota(jnp.int32, (256, 128), 1)
    src = 32 * (c_idx // 16) + (c_idx % 16)
    phi = (k_idx == src + 16).astype(jnp.float32)
    plo = (k_idx == src).astype(jnp.float32)
    tab_u = _repack(user_table.T, phi, plo)
    tab_m = _repack(movie_table.T, phi, plo)
    g = gather_kernel(uq, mq, tab_u, tab_m)

    ur = ((uid % SPAN) // REPACK_BLK).reshape(BATCH, 1)
    mr = ((mid % SPAN) // REPACK_BLK).reshape(BATCH, 1)
    W1s = (W1[0:16], W1[16:32], W1[32:48], W1[48:64])
    return _mlp_call(g, ur, mr, W1s, b1.reshape(1, 256), W2,
                     b2.reshape(1, 64), W3.reshape(1, 64), b3.reshape(1, 1),
                     blk=1024)


# speedup vs baseline: 1.9629x; 1.0203x over previous
import functools

import jax
import jax.numpy as jnp
from jax import lax
from jax.experimental import pallas as pl
from jax.experimental.pallas import tpu as pltpu
from jax.experimental.pallas import tpu_sc as plsc

BATCH = 16384
EMBED = 32
PACKED = 128
ROWS_PER = 8
VOCAB = 1000000
IDX_CHUNK = 128

REPACK_BLK = 16384
SPAN = ROWS_PER * REPACK_BLK
N_BLOCKS = -(-VOCAB // SPAN)
VQ_PAD = N_BLOCKS * REPACK_BLK


def _make_gather(num_cores: int, num_workers: int):
    b_per_w = BATCH // num_workers
    n_chunks = b_per_w // IDX_CHUNK
    mesh = plsc.VectorSubcoreMesh(core_axis_name="c", subcore_axis_name="s")

    @functools.partial(
        pl.kernel,
        mesh=mesh,
        compiler_params=pltpu.CompilerParams(use_tc_tiling_on_sc=False),
        out_type=jax.ShapeDtypeStruct((BATCH, 2 * PACKED), jnp.float32),
        scratch_types=[
            pltpu.VMEM((n_chunks, IDX_CHUNK), jnp.int32),
            pltpu.VMEM((n_chunks, IDX_CHUNK), jnp.int32),
            pltpu.VMEM((b_per_w // 2, PACKED), jnp.float32),
            pltpu.VMEM((b_per_w // 2, PACKED), jnp.float32),
            pltpu.SemaphoreType.DMA,
        ],
    )
    def gather_kernel(uid_hbm, mid_hbm, utab_hbm, mtab_hbm, out_hbm,
                      uidx_v, midx_v, urows_v, mrows_v, sem):
        wid = lax.axis_index("s") * num_cores + lax.axis_index("c")
        base = wid * b_per_w
        half = b_per_w // 2
        c_per_p = n_chunks // 2
        pltpu.sync_copy(uid_hbm.at[wid], uidx_v)
        pltpu.sync_copy(mid_hbm.at[wid], midx_v)
        for p in range(2):
            copies = []
            for tab_hbm, idx_v, rows_v in ((utab_hbm, uidx_v, urows_v),
                                           (mtab_hbm, midx_v, mrows_v)):
                for j in range(c_per_p):
                    copies.append(pltpu.async_copy(
                        tab_hbm.at[idx_v.at[p * c_per_p + j]],
                        rows_v.at[pl.ds(j * IDX_CHUNK, IDX_CHUNK)], sem))
            for c in copies:
                c.wait()
            row0 = base + p * half
            pltpu.sync_copy(urows_v,
                            out_hbm.at[pl.ds(row0, half), pl.ds(0, PACKED)])
            pltpu.sync_copy(mrows_v,
                            out_hbm.at[pl.ds(row0, half), pl.ds(PACKED, PACKED)])

    return gather_kernel, n_chunks


def _pack_pair(hi, lo):
    hb = lax.bitcast_convert_type(hi, jnp.uint32)
    lb = lax.bitcast_convert_type(lo, jnp.uint32)
    hb = (hb + jnp.uint32(0x8000)) & jnp.uint32(0xFFFF0000)
    lb = (lb + jnp.uint32(0x8000)) >> jnp.uint32(16)
    return lax.bitcast_convert_type(hb | lb, jnp.float32)


def _repack_body(t_ref, phi_ref, plo_ref, o_ref):
    x = t_ref[...]
    n = o_ref.shape[0]
    x8 = jnp.concatenate([x[:, n * a:n * (a + 1)] for a in range(8)],
                         axis=0)
    y_hi = jax.lax.dot_general(
        x8, phi_ref[...], (((0,), (0,)), ((), ())),
        preferred_element_type=jnp.float32)
    y_lo = jax.lax.dot_general(
        x8, plo_ref[...], (((0,), (0,)), ((), ())),
        preferred_element_type=jnp.float32)
    o_ref[...] = _pack_pair(y_hi, y_lo)


def _repack(table_t, phi, plo):
    return pl.pallas_call(
        _repack_body,
        grid=(N_BLOCKS,),
        in_specs=[
            pl.BlockSpec((EMBED, SPAN), lambda i: (0, i)),
            pl.BlockSpec((256, 128), lambda i: (0, 0)),
            pl.BlockSpec((256, 128), lambda i: (0, 0)),
        ],
        out_specs=pl.BlockSpec((REPACK_BLK, PACKED), lambda i: (i, 0)),
        out_shape=jax.ShapeDtypeStruct((VQ_PAD, PACKED), jnp.float32),
    )(table_t, phi, plo)


def _select16(rows, rem, off):
    w = [rows[:, off + 16 * a:off + 16 * (a + 1)] for a in range(8)]
    b0 = (rem & 1) == 0
    b1 = (rem & 2) == 0
    l1 = [jnp.where(b0, w[2 * a], w[2 * a + 1]) for a in range(4)]
    l2 = [jnp.where(b1, l1[2 * a], l1[2 * a + 1]) for a in range(2)]
    return jnp.where(rem < 4, l2[0], l2[1])


def _unpack(sel):
    w = lax.bitcast_convert_type(sel, jnp.uint32)
    lo = lax.bitcast_convert_type(w << jnp.uint32(16), jnp.float32)
    hi = lax.bitcast_convert_type(w & jnp.uint32(0xFFFF0000), jnp.float32)
    return lo, hi


def _mlp_body(g_ref, ur_ref, mr_ref, w1a_ref, w1b_ref, w1c_ref, w1d_ref,
              b1_ref, w2_ref, b2_ref, w3_ref, b3_ref, o_ref):
    g = g_ref[...]
    ulo, uhi = _unpack(_select16(g, ur_ref[...], 0))
    mlo, mhi = _unpack(_select16(g, mr_ref[...], PACKED))
    f32 = jnp.float32
    h1 = jnp.dot(ulo, w1a_ref[...], preferred_element_type=f32)
    h1 += jnp.dot(uhi, w1b_ref[...], preferred_element_type=f32)
    h1 += jnp.dot(mlo, w1c_ref[...], preferred_element_type=f32)
    h1 += jnp.dot(mhi, w1d_ref[...], preferred_element_type=f32)
    h1 = jnp.maximum(h1 + b1_ref[...], 0.0)
    h2 = jnp.dot(h1, w2_ref[...], preferred_element_type=f32)
    h2 = jnp.maximum(h2 + b2_ref[...], 0.0)
    o_ref[...] = jnp.sum(h2 * w3_ref[...], axis=1, keepdims=True) + b3_ref[...]


def _mlp_call(g, ur, mr, W1s, b1, W2, b2, W3r, b3, blk: int):
    grid = (BATCH // blk,)
    return pl.pallas_call(
        _mlp_body,
        grid=grid,
        in_specs=[
            pl.BlockSpec((blk, 2 * PACKED), lambda i: (i, 0)),
            pl.BlockSpec((blk, 1), lambda i: (i, 0)),
            pl.BlockSpec((blk, 1), lambda i: (i, 0)),
            pl.BlockSpec((16, 256), lambda i: (0, 0)),
            pl.BlockSpec((16, 256), lambda i: (0, 0)),
            pl.BlockSpec((16, 256), lambda i: (0, 0)),
            pl.BlockSpec((16, 256), lambda i: (0, 0)),
            pl.BlockSpec((1, 256), lambda i: (0, 0)),
            pl.BlockSpec((256, 64), lambda i: (0, 0)),
            pl.BlockSpec((1, 64), lambda i: (0, 0)),
            pl.BlockSpec((1, 64), lambda i: (0, 0)),
            pl.BlockSpec((1, 1), lambda i: (0, 0)),
        ],
        out_specs=pl.BlockSpec((blk, 1), lambda i: (i, 0)),
        out_shape=jax.ShapeDtypeStruct((BATCH, 1), jnp.float32),
    )(g, ur, mr, *W1s, b1, W2, b2, W3r, b3)


def kernel(userId, movieId, user_table, movie_table, W1, b1, W2, b2, W3, b3):
    info = plsc.get_sparse_core_info()
    num_workers = info.num_cores * info.num_subcores
    gather_kernel, n_chunks = _make_gather(info.num_cores, num_workers)

    uid = userId.astype(jnp.int32)
    mid = movieId.astype(jnp.int32)
    uq = (REPACK_BLK * (uid // SPAN) + (uid % SPAN) % REPACK_BLK)
    mq = (REPACK_BLK * (mid // SPAN) + (mid % SPAN) % REPACK_BLK)
    uq = uq.reshape(num_workers, n_chunks, IDX_CHUNK)
    mq = mq.reshape(num_workers, n_chunks, IDX_CHUNK)
    k_idx = lax.broadcasted_iota(jnp.int32, (256, 128), 0)
    c_idx = lax.broadcasted_iota(jnp.int32, (256, 128), 1)
    src = 32 * (c_idx // 16) + (c_idx % 16)
    phi = (k_idx == src + 16).astype(jnp.float32)
    plo = (k_idx == src).astype(jnp.float32)
    tab_u = _repack(user_table.T, phi, plo)
    tab_m = _repack(movie_table.T, phi, plo)
    g = gather_kernel(uq, mq, tab_u, tab_m)

    ur = ((uid % SPAN) // REPACK_BLK).reshape(BATCH, 1)
    mr = ((mid % SPAN) // REPACK_BLK).reshape(BATCH, 1)
    W1s = (W1[0:16], W1[16:32], W1[32:48], W1[48:64])
    return _mlp_call(g, ur, mr, W1s, b1.reshape(1, 256), W2,
                     b2.reshape(1, 64), W3.reshape(1, 64), b3.reshape(1, 1),
                     blk=1024)


# speedup vs baseline: 2.1499x; 1.0953x over previous
import functools

import jax
import jax.numpy as jnp
from jax import lax
from jax.experimental import pallas as pl
from jax.experimental.pallas import tpu as pltpu
from jax.experimental.pallas import tpu_sc as plsc

BATCH = 16384
EMBED = 32
PACKED = 128
ROWS_PER = 8
VOCAB = 1000000
IDX_CHUNK = 128

REPACK_BLK = 16384
SPAN = ROWS_PER * REPACK_BLK
N_BLOCKS = -(-VOCAB // SPAN)
VQ_PAD = N_BLOCKS * REPACK_BLK


def _make_gather(num_cores: int, num_workers: int):
    b_per_w = BATCH // num_workers
    n_chunks = b_per_w // IDX_CHUNK
    mesh = plsc.VectorSubcoreMesh(core_axis_name="c", subcore_axis_name="s")

    @functools.partial(
        pl.kernel,
        mesh=mesh,
        compiler_params=pltpu.CompilerParams(use_tc_tiling_on_sc=False),
        out_type=jax.ShapeDtypeStruct((BATCH, PACKED), jnp.float32),
        scratch_types=[
            pltpu.VMEM((n_chunks, IDX_CHUNK), jnp.int32),
            pltpu.VMEM((b_per_w, PACKED), jnp.float32),
            pltpu.SemaphoreType.DMA,
        ],
    )
    def gather_kernel(id_hbm, tab_hbm, out_hbm, idx_v, rows_v, sem):
        wid = lax.axis_index("s") * num_cores + lax.axis_index("c")
        base = wid * b_per_w
        pltpu.sync_copy(id_hbm.at[wid], idx_v)
        copies = []
        for j in range(n_chunks):
            copies.append(pltpu.async_copy(
                tab_hbm.at[idx_v.at[j]],
                rows_v.at[pl.ds(j * IDX_CHUNK, IDX_CHUNK)], sem))
        for c in copies:
            c.wait()
        pltpu.sync_copy(rows_v, out_hbm.at[pl.ds(base, b_per_w)])

    return gather_kernel, n_chunks


def _pack_pair(hi, lo):
    hb = lax.bitcast_convert_type(hi, jnp.uint32)
    lb = lax.bitcast_convert_type(lo, jnp.uint32)
    hb = (hb + jnp.uint32(0x8000)) & jnp.uint32(0xFFFF0000)
    lb = (lb + jnp.uint32(0x8000)) >> jnp.uint32(16)
    return lax.bitcast_convert_type(hb | lb, jnp.float32)


def _repack_body(t_ref, phi_ref, plo_ref, o_ref):
    x = t_ref[...]
    n = o_ref.shape[0]
    x8 = jnp.concatenate([x[:, n * a:n * (a + 1)] for a in range(8)],
                         axis=0)
    y_hi = jax.lax.dot_general(
        x8, phi_ref[...], (((0,), (0,)), ((), ())),
        preferred_element_type=jnp.float32)
    y_lo = jax.lax.dot_general(
        x8, plo_ref[...], (((0,), (0,)), ((), ())),
        preferred_element_type=jnp.float32)
    o_ref[...] = _pack_pair(y_hi, y_lo)


def _repack(table_t, phi, plo):
    return pl.pallas_call(
        _repack_body,
        grid=(N_BLOCKS,),
        in_specs=[
            pl.BlockSpec((EMBED, SPAN), lambda i: (0, i)),
            pl.BlockSpec((256, 128), lambda i: (0, 0)),
            pl.BlockSpec((256, 128), lambda i: (0, 0)),
        ],
        out_specs=pl.BlockSpec((REPACK_BLK, PACKED), lambda i: (i, 0)),
        out_shape=jax.ShapeDtypeStruct((VQ_PAD, PACKED), jnp.float32),
    )(table_t, phi, plo)


def _select16(rows, rem):
    w = [rows[:, 16 * a:16 * (a + 1)] for a in range(8)]
    b0 = (rem & 1) == 0
    b1 = (rem & 2) == 0
    l1 = [jnp.where(b0, w[2 * a], w[2 * a + 1]) for a in range(4)]
    l2 = [jnp.where(b1, l1[2 * a], l1[2 * a + 1]) for a in range(2)]
    return jnp.where(rem < 4, l2[0], l2[1])


def _unpack(sel):
    w = lax.bitcast_convert_type(sel, jnp.uint32)
    lo = lax.bitcast_convert_type(w << jnp.uint32(16), jnp.float32)
    hi = lax.bitcast_convert_type(w & jnp.uint32(0xFFFF0000), jnp.float32)
    return lo, hi


def _mlp_body(gu_ref, gm_ref, ur_ref, mr_ref, w1a_ref, w1b_ref, w1c_ref,
              w1d_ref, b1_ref, w2_ref, b2_ref, w3_ref, b3_ref, o_ref):
    ulo, uhi = _unpack(_select16(gu_ref[...], ur_ref[...]))
    mlo, mhi = _unpack(_select16(gm_ref[...], mr_ref[...]))
    f32 = jnp.float32
    h1 = jnp.dot(ulo, w1a_ref[...], preferred_element_type=f32)
    h1 += jnp.dot(uhi, w1b_ref[...], preferred_element_type=f32)
    h1 += jnp.dot(mlo, w1c_ref[...], preferred_element_type=f32)
    h1 += jnp.dot(mhi, w1d_ref[...], preferred_element_type=f32)
    h1 = jnp.maximum(h1 + b1_ref[...], 0.0)
    h2 = jnp.dot(h1, w2_ref[...], preferred_element_type=f32)
    h2 = jnp.maximum(h2 + b2_ref[...], 0.0)
    o_ref[...] = jnp.sum(h2 * w3_ref[...], axis=1, keepdims=True) + b3_ref[...]


def _mlp_call(gu, gm, ur, mr, W1s, b1, W2, b2, W3r, b3, blk: int):
    grid = (BATCH // blk,)
    return pl.pallas_call(
        _mlp_body,
        grid=grid,
        in_specs=[
            pl.BlockSpec((blk, PACKED), lambda i: (i, 0)),
            pl.BlockSpec((blk, PACKED), lambda i: (i, 0)),
            pl.BlockSpec((blk, 1), lambda i: (i, 0)),
            pl.BlockSpec((blk, 1), lambda i: (i, 0)),
            pl.BlockSpec((16, 256), lambda i: (0, 0)),
            pl.BlockSpec((16, 256), lambda i: (0, 0)),
            pl.BlockSpec((16, 256), lambda i: (0, 0)),
            pl.BlockSpec((16, 256), lambda i: (0, 0)),
            pl.BlockSpec((1, 256), lambda i: (0, 0)),
            pl.BlockSpec((256, 64), lambda i: (0, 0)),
            pl.BlockSpec((1, 64), lambda i: (0, 0)),
            pl.BlockSpec((1, 64), lambda i: (0, 0)),
            pl.BlockSpec((1, 1), lambda i: (0, 0)),
        ],
        out_specs=pl.BlockSpec((blk, 1), lambda i: (i, 0)),
        out_shape=jax.ShapeDtypeStruct((BATCH, 1), jnp.float32),
    )(gu, gm, ur, mr, *W1s, b1, W2, b2, W3r, b3)


def kernel(userId, movieId, user_table, movie_table, W1, b1, W2, b2, W3, b3):
    info = plsc.get_sparse_core_info()
    num_workers = info.num_cores * info.num_subcores
    gather_kernel, n_chunks = _make_gather(info.num_cores, num_workers)

    uid = userId.astype(jnp.int32)
    mid = movieId.astype(jnp.int32)
    uq = (REPACK_BLK * (uid // SPAN) + (uid % SPAN) % REPACK_BLK)
    mq = (REPACK_BLK * (mid // SPAN) + (mid % SPAN) % REPACK_BLK)
    uq = uq.reshape(num_workers, n_chunks, IDX_CHUNK)
    mq = mq.reshape(num_workers, n_chunks, IDX_CHUNK)
    k_idx = lax.broadcasted_iota(jnp.int32, (256, 128), 0)
    c_idx = lax.broadcasted_iota(jnp.int32, (256, 128), 1)
    src = 32 * (c_idx // 16) + (c_idx % 16)
    phi = (k_idx == src + 16).astype(jnp.float32)
    plo = (k_idx == src).astype(jnp.float32)
    tab_u = _repack(user_table.T, phi, plo)
    gu = gather_kernel(uq, tab_u)
    tab_m = _repack(movie_table.T, phi, plo)
    gm = gather_kernel(mq, tab_m)

    ur = ((uid % SPAN) // REPACK_BLK).reshape(BATCH, 1)
    mr = ((mid % SPAN) // REPACK_BLK).reshape(BATCH, 1)
    W1s = (W1[0:16], W1[16:32], W1[32:48], W1[48:64])
    return _mlp_call(gu, gm, ur, mr, W1s, b1.reshape(1, 256), W2,
                     b2.reshape(1, 64), W3.reshape(1, 64), b3.reshape(1, 1),
                     blk=1024)


# speedup vs baseline: 2.3858x; 1.1097x over previous
import functools

import jax
import jax.numpy as jnp
from jax import lax
from jax.experimental import pallas as pl
from jax.experimental.pallas import tpu as pltpu
from jax.experimental.pallas import tpu_sc as plsc

BATCH = 16384
EMBED = 32
PACKED = 128
ROWS_PER = 8
SUB_W = PACKED // ROWS_PER
VOCAB = 1000000
IDX_CHUNK = 128

REPACK_BLK = 16384
SPAN = ROWS_PER * REPACK_BLK
N_BLOCKS = -(-VOCAB // SPAN)
VQ_PAD = N_BLOCKS * REPACK_BLK


def _make_gather(num_cores: int, num_workers: int):
    b_per_w = BATCH // num_workers
    n_chunks = b_per_w // IDX_CHUNK
    mesh = plsc.VectorSubcoreMesh(core_axis_name="c", subcore_axis_name="s")

    @functools.partial(
        pl.kernel,
        mesh=mesh,
        compiler_params=pltpu.CompilerParams(use_tc_tiling_on_sc=False),
        out_type=jax.ShapeDtypeStruct((BATCH, SUB_W), jnp.float32),
        scratch_types=[
            pltpu.VMEM((n_chunks, IDX_CHUNK), jnp.int32),
            pltpu.VMEM((b_per_w, SUB_W), jnp.float32),
            pltpu.SemaphoreType.DMA,
        ],
    )
    def gather_kernel(id_hbm, tab_hbm, out_hbm, idx_v, rows_v, sem):
        wid = lax.axis_index("s") * num_cores + lax.axis_index("c")
        base = wid * b_per_w
        pltpu.sync_copy(id_hbm.at[wid], idx_v)
        copies = []
        for j in range(n_chunks):
            copies.append(pltpu.async_copy(
                tab_hbm.at[idx_v.at[j]],
                rows_v.at[pl.ds(j * IDX_CHUNK, IDX_CHUNK)], sem))
        for c in copies:
            c.wait()
        pltpu.sync_copy(rows_v, out_hbm.at[pl.ds(base, b_per_w)])

    return gather_kernel, n_chunks


def _pack_pair(hi, lo):
    hb = lax.bitcast_convert_type(hi, jnp.uint32)
    lb = lax.bitcast_convert_type(lo, jnp.uint32)
    hb = (hb + jnp.uint32(0x8000)) & jnp.uint32(0xFFFF0000)
    lb = (lb + jnp.uint32(0x8000)) >> jnp.uint32(16)
    return lax.bitcast_convert_type(hb | lb, jnp.float32)


def _repack_body(t_ref, phi_ref, plo_ref, o_ref):
    x = t_ref[...]
    n = o_ref.shape[0]
    x8 = jnp.concatenate([x[:, n * a:n * (a + 1)] for a in range(8)],
                         axis=0)
    y_hi = jax.lax.dot_general(
        x8, phi_ref[...], (((0,), (0,)), ((), ())),
        preferred_element_type=jnp.float32)
    y_lo = jax.lax.dot_general(
        x8, plo_ref[...], (((0,), (0,)), ((), ())),
        preferred_element_type=jnp.float32)
    o_ref[...] = _pack_pair(y_hi, y_lo)


def _repack(table_t, phi, plo):
    return pl.pallas_call(
        _repack_body,
        grid=(N_BLOCKS,),
        in_specs=[
            pl.BlockSpec((EMBED, SPAN), lambda i: (0, i)),
            pl.BlockSpec((256, 128), lambda i: (0, 0)),
            pl.BlockSpec((256, 128), lambda i: (0, 0)),
        ],
        out_specs=pl.BlockSpec((REPACK_BLK, PACKED), lambda i: (i, 0)),
        out_shape=jax.ShapeDtypeStruct((VQ_PAD, PACKED), jnp.float32),
    )(table_t, phi, plo)


def _unpack(sel):
    w = lax.bitcast_convert_type(sel, jnp.uint32)
    lo = lax.bitcast_convert_type(w << jnp.uint32(16), jnp.float32)
    hi = lax.bitcast_convert_type(w & jnp.uint32(0xFFFF0000), jnp.float32)
    return lo, hi


def _mlp_body(gu_ref, gm_ref, w1a_ref, w1b_ref, w1c_ref,
              w1d_ref, b1_ref, w2_ref, b2_ref, w3_ref, b3_ref, o_ref):
    ulo, uhi = _unpack(gu_ref[...])
    mlo, mhi = _unpack(gm_ref[...])
    f32 = jnp.float32
    h1 = jnp.dot(ulo, w1a_ref[...], preferred_element_type=f32)
    h1 += jnp.dot(uhi, w1b_ref[...], preferred_element_type=f32)
    h1 += jnp.dot(mlo, w1c_ref[...], preferred_element_type=f32)
    h1 += jnp.dot(mhi, w1d_ref[...], preferred_element_type=f32)
    h1 = jnp.maximum(h1 + b1_ref[...], 0.0)
    h2 = jnp.dot(h1, w2_ref[...], preferred_element_type=f32)
    h2 = jnp.maximum(h2 + b2_ref[...], 0.0)
    o_ref[...] = jnp.sum(h2 * w3_ref[...], axis=1, keepdims=True) + b3_ref[...]


def _mlp_call(gu, gm, W1s, b1, W2, b2, W3r, b3, blk: int):
    grid = (BATCH // blk,)
    return pl.pallas_call(
        _mlp_body,
        grid=grid,
        in_specs=[
            pl.BlockSpec((blk, SUB_W), lambda i: (i, 0)),
            pl.BlockSpec((blk, SUB_W), lambda i: (i, 0)),
            pl.BlockSpec((16, 256), lambda i: (0, 0)),
            pl.BlockSpec((16, 256), lambda i: (0, 0)),
            pl.BlockSpec((16, 256), lambda i: (0, 0)),
            pl.BlockSpec((16, 256), lambda i: (0, 0)),
            pl.BlockSpec((1, 256), lambda i: (0, 0)),
            pl.BlockSpec((256, 64), lambda i: (0, 0)),
            pl.BlockSpec((1, 64), lambda i: (0, 0)),
            pl.BlockSpec((1, 64), lambda i: (0, 0)),
            pl.BlockSpec((1, 1), lambda i: (0, 0)),
        ],
        out_specs=pl.BlockSpec((blk, 1), lambda i: (i, 0)),
        out_shape=jax.ShapeDtypeStruct((BATCH, 1), jnp.float32),
    )(gu, gm, *W1s, b1, W2, b2, W3r, b3)


def kernel(userId, movieId, user_table, movie_table, W1, b1, W2, b2, W3, b3):
    info = plsc.get_sparse_core_info()
    num_workers = info.num_cores * info.num_subcores
    gather_kernel, n_chunks = _make_gather(info.num_cores, num_workers)

    uid = userId.astype(jnp.int32)
    mid = movieId.astype(jnp.int32)
    uq = (ROWS_PER * (REPACK_BLK * (uid // SPAN) + (uid % SPAN) % REPACK_BLK)
          + (uid % SPAN) // REPACK_BLK)
    mq = (ROWS_PER * (REPACK_BLK * (mid // SPAN) + (mid % SPAN) % REPACK_BLK)
          + (mid % SPAN) // REPACK_BLK)
    uq = uq.reshape(num_workers, n_chunks, IDX_CHUNK)
    mq = mq.reshape(num_workers, n_chunks, IDX_CHUNK)
    k_idx = lax.broadcasted_iota(jnp.int32, (256, 128), 0)
    c_idx = lax.broadcasted_iota(jnp.int32, (256, 128), 1)
    src = 32 * (c_idx // 16) + (c_idx % 16)
    phi = (k_idx == src + 16).astype(jnp.float32)
    plo = (k_idx == src).astype(jnp.float32)
    tab_u = _repack(user_table.T, phi, plo).reshape(VQ_PAD * ROWS_PER, SUB_W)
    gu = gather_kernel(uq, tab_u)
    tab_m = _repack(movie_table.T, phi, plo).reshape(VQ_PAD * ROWS_PER, SUB_W)
    gm = gather_kernel(mq, tab_m)

    W1s = (W1[0:16], W1[16:32], W1[32:48], W1[48:64])
    return _mlp_call(gu, gm, W1s, b1.reshape(1, 256), W2,
                     b2.reshape(1, 64), W3.reshape(1, 64), b3.reshape(1, 1),
                     blk=1024)


# speedup vs baseline: 2.4606x; 1.0314x over previous
import functools

import jax
import jax.numpy as jnp
from jax import lax
from jax.experimental import pallas as pl
from jax.experimental.pallas import tpu as pltpu
from jax.experimental.pallas import tpu_sc as plsc

BATCH = 16384
EMBED = 32
PACKED = 128
ROWS_PER = 8
SUB_W = PACKED // ROWS_PER
VOCAB = 1000000
IDX_CHUNK = 128

REPACK_BLK = 16384
SPAN = ROWS_PER * REPACK_BLK
N_BLOCKS = -(-VOCAB // SPAN)
VQ_PAD = N_BLOCKS * REPACK_BLK


def _make_gather(num_cores: int, num_workers: int):
    b_per_w = BATCH // num_workers
    n_chunks = b_per_w // IDX_CHUNK
    mesh = plsc.VectorSubcoreMesh(core_axis_name="c", subcore_axis_name="s")

    @functools.partial(
        pl.kernel,
        mesh=mesh,
        compiler_params=pltpu.CompilerParams(use_tc_tiling_on_sc=False),
        out_type=jax.ShapeDtypeStruct((BATCH, SUB_W), jnp.float32),
        scratch_types=[
            pltpu.VMEM((n_chunks, IDX_CHUNK), jnp.int32),
            pltpu.VMEM((b_per_w, SUB_W), jnp.float32),
            pltpu.SemaphoreType.DMA,
        ],
    )
    def gather_kernel(id_hbm, tab_hbm, out_hbm, idx_v, rows_v, sem):
        wid = lax.axis_index("s") * num_cores + lax.axis_index("c")
        base = wid * b_per_w
        pltpu.sync_copy(id_hbm.at[wid], idx_v)
        copies = []
        for j in range(n_chunks):
            copies.append(pltpu.async_copy(
                tab_hbm.at[idx_v.at[j]],
                rows_v.at[pl.ds(j * IDX_CHUNK, IDX_CHUNK)], sem))
        for c in copies:
            c.wait()
        pltpu.sync_copy(rows_v, out_hbm.at[pl.ds(base, b_per_w)])

    return gather_kernel, n_chunks


def _pack_pair(hi, lo):
    hb = lax.bitcast_convert_type(hi, jnp.uint32)
    lb = lax.bitcast_convert_type(lo, jnp.uint32)
    hb = (hb + jnp.uint32(0x8000)) & jnp.uint32(0xFFFF0000)
    lb = (lb + jnp.uint32(0x8000)) >> jnp.uint32(16)
    return lax.bitcast_convert_type(hb | lb, jnp.float32)


def _repack_body(t_ref, phi_ref, plo_ref, o_ref):
    x = t_ref[...]
    n = o_ref.shape[0]
    x8 = jnp.concatenate([x[:, n * a:n * (a + 1)] for a in range(8)],
                         axis=0)
    y_hi = jax.lax.dot_general(
        x8, phi_ref[...], (((0,), (0,)), ((), ())),
        preferred_element_type=jnp.float32)
    y_lo = jax.lax.dot_general(
        x8, plo_ref[...], (((0,), (0,)), ((), ())),
        preferred_element_type=jnp.float32)
    o_ref[...] = _pack_pair(y_hi, y_lo)


def _repack(table_t, phi, plo):
    return pl.pallas_call(
        _repack_body,
        grid=(N_BLOCKS,),
        in_specs=[
            pl.BlockSpec((EMBED, SPAN), lambda i: (0, i)),
            pl.BlockSpec((256, 128), lambda i: (0, 0)),
            pl.BlockSpec((256, 128), lambda i: (0, 0)),
        ],
        out_specs=pl.BlockSpec((REPACK_BLK, PACKED), lambda i: (i, 0)),
        out_shape=jax.ShapeDtypeStruct((VQ_PAD, PACKED), jnp.float32),
    )(table_t, phi, plo)


def _unpack(sel):
    w = lax.bitcast_convert_type(sel, jnp.uint32)
    lo = lax.bitcast_convert_type(w << jnp.uint32(16), jnp.float32)
    hi = lax.bitcast_convert_type(w & jnp.uint32(0xFFFF0000), jnp.float32)
    return lo, hi


def _mlp_body(gu_ref, gm_ref, w1a_ref, w1b_ref, w1c_ref,
              w1d_ref, b1_ref, w2_ref, b2_ref, w3_ref, b3_ref, o_ref):
    ulo, uhi = _unpack(gu_ref[...])
    mlo, mhi = _unpack(gm_ref[...])
    f32 = jnp.float32
    h1 = jnp.dot(ulo, w1a_ref[...], preferred_element_type=f32)
    h1 += jnp.dot(uhi, w1b_ref[...], preferred_element_type=f32)
    h1 += jnp.dot(mlo, w1c_ref[...], preferred_element_type=f32)
    h1 += jnp.dot(mhi, w1d_ref[...], preferred_element_type=f32)
    h1 = jnp.maximum(h1 + b1_ref[...], 0.0)
    h2 = jnp.dot(h1, w2_ref[...], preferred_element_type=f32)
    h2 = jnp.maximum(h2 + b2_ref[...], 0.0)
    o_ref[...] = jnp.sum(h2 * w3_ref[...], axis=1, keepdims=True) + b3_ref[...]


def _mlp_call(gu, gm, W1s, b1, W2, b2, W3r, b3, blk: int):
    grid = (BATCH // blk,)
    return pl.pallas_call(
        _mlp_body,
        grid=grid,
        in_specs=[
            pl.BlockSpec((blk, SUB_W), lambda i: (i, 0)),
            pl.BlockSpec((blk, SUB_W), lambda i: (i, 0)),
            pl.BlockSpec((16, 256), lambda i: (0, 0)),
            pl.BlockSpec((16, 256), lambda i: (0, 0)),
            pl.BlockSpec((16, 256), lambda i: (0, 0)),
            pl.BlockSpec((16, 256), lambda i: (0, 0)),
            pl.BlockSpec((1, 256), lambda i: (0, 0)),
            pl.BlockSpec((256, 64), lambda i: (0, 0)),
            pl.BlockSpec((1, 64), lambda i: (0, 0)),
            pl.BlockSpec((1, 64), lambda i: (0, 0)),
            pl.BlockSpec((1, 1), lambda i: (0, 0)),
        ],
        out_specs=pl.BlockSpec((blk, 1), lambda i: (i, 0)),
        out_shape=jax.ShapeDtypeStruct((BATCH, 1), jnp.float32),
    )(gu, gm, *W1s, b1, W2, b2, W3r, b3)


def kernel(userId, movieId, user_table, movie_table, W1, b1, W2, b2, W3, b3):
    info = plsc.get_sparse_core_info()
    num_workers = info.num_cores * info.num_subcores
    gather_kernel, n_chunks = _make_gather(info.num_cores, num_workers)

    uid = userId.astype(jnp.int32)
    mid = movieId.astype(jnp.int32)
    uq = (ROWS_PER * (REPACK_BLK * (uid // SPAN) + (uid % SPAN) % REPACK_BLK)
          + (uid % SPAN) // REPACK_BLK)
    mq = (ROWS_PER * (REPACK_BLK * (mid // SPAN) + (mid % SPAN) % REPACK_BLK)
          + (mid % SPAN) // REPACK_BLK)
    uq = uq.reshape(num_workers, n_chunks, IDX_CHUNK)
    mq = mq.reshape(num_workers, n_chunks, IDX_CHUNK)
    k_idx = lax.broadcasted_iota(jnp.int32, (256, 128), 0)
    c_idx = lax.broadcasted_iota(jnp.int32, (256, 128), 1)
    src = 32 * (c_idx // 16) + (c_idx % 16)
    phi = (k_idx == src + 16).astype(jnp.float32)
    plo = (k_idx == src).astype(jnp.float32)
    tab_u = _repack(user_table.T, phi, plo).reshape(VQ_PAD * ROWS_PER, SUB_W)
    gu = gather_kernel(uq, tab_u)
    tab_m = _repack(movie_table.T, phi, plo).reshape(VQ_PAD * ROWS_PER, SUB_W)
    gm = gather_kernel(mq, tab_m)

    W1s = (W1[0:16], W1[16:32], W1[32:48], W1[48:64])
    return _mlp_call(gu, gm, W1s, b1.reshape(1, 256), W2,
                     b2.reshape(1, 64), W3.reshape(1, 64), b3.reshape(1, 1),
                     blk=4096)
